# baseline (speedup 1.0000x reference)
import functools

import jax
import jax.numpy as jnp
from jax import lax
from jax.experimental import pallas as pl
from jax.experimental.pallas import tpu as pltpu
from jax.experimental.pallas import tpu_sc as plsc

EMB = 32
BATCH = 16384
VOCAB_A = 1000000
VOCAB_B = 100000
CHUNK = 128
LANES = 16


@functools.cache
def _build():
    info = plsc.get_sparse_core_info()
    nw = info.num_cores * info.num_subcores
    n = BATCH // nw
    nch = n // CHUNK
    blk = 2 * n * EMB

    mesh = plsc.VectorSubcoreMesh(core_axis_name="c", subcore_axis_name="s")

    @functools.partial(
        pl.kernel,
        mesh=mesh,
        out_type=jax.ShapeDtypeStruct((2 * BATCH * EMB,), jnp.float32),
        compiler_params=pltpu.CompilerParams(
            use_tc_tiling_on_sc=False, needs_layout_passes=False
        ),
        scratch_types=[
            pltpu.VMEM((n,), jnp.int32),
            pltpu.VMEM((n,), jnp.int32),
            pltpu.VMEM((blk,), jnp.int32),
            pltpu.VMEM((blk,), jnp.float32),
            pltpu.SemaphoreType.DMA,
        ],
    )
    def k(idx_a_hbm, idx_b_hbm, wa_hbm, wb_hbm, out_hbm,
          ia_v, ib_v, fi_v, o_v, sem):
        wid = lax.axis_index("s") * info.num_cores + lax.axis_index("c")
        cp_a = pltpu.async_copy(idx_a_hbm.at[pl.ds(wid * n, n)], ia_v, sem)
        cp_b = pltpu.async_copy(idx_b_hbm.at[pl.ds(wid * n, n)], ib_v, sem)
        iot = lax.iota(jnp.int32, LANES)
        base_lane = iot * (2 * EMB)
        cp_a.wait()
        cp_b.wait()

        def build(g, carry):
            ids_a = ia_v[pl.ds(g * LANES, LANES)]
            ids_b = ib_v[pl.ds(g * LANES, LANES)]
            pos0 = base_lane + g * (LANES * 2 * EMB)
            for f in range(EMB):
                plsc.store_scatter(fi_v, [pos0 + f], ids_a + f * VOCAB_A)
                plsc.store_scatter(fi_v, [pos0 + (EMB + f)], ids_b + f * VOCAB_B)
            return carry

        lax.fori_loop(0, n // LANES, build, 0)

        def fire_a(r, carry):
            base = 2 * EMB * r
            pltpu.async_copy(
                wa_hbm.at[fi_v.at[pl.ds(base, EMB)]], o_v.at[pl.ds(base, EMB)], sem
            )
            return carry

        def fire_b(r, carry):
            base = 2 * EMB * r + EMB
            pltpu.async_copy(
                wb_hbm.at[fi_v.at[pl.ds(base, EMB)]], o_v.at[pl.ds(base, EMB)], sem
            )
            return carry

        lax.fori_loop(0, n, fire_a, 0)
        lax.fori_loop(0, n, fire_b, 0)
        pltpu.make_async_copy(wa_hbm.at[pl.ds(0, blk)], o_v, sem).wait()
        pltpu.sync_copy(o_v, out_hbm.at[pl.ds(wid * blk, blk)])

    return k


def kernel(field_a, field_b, W_a, W_b):
    k = _build()
    ia = field_a
    ib = field_b
    wa_flat = lax.optimization_barrier(W_a.T).reshape(-1)
    wb_flat = lax.optimization_barrier(W_b.T).reshape(-1)
    out1 = k(ia, ib, wa_flat, wb_flat)
    return out1.reshape(BATCH, 2 * EMB)


# speedup vs baseline: 4.7960x; 4.7960x over previous
import functools

import jax
import jax.numpy as jnp
from jax import lax
from jax.experimental import pallas as pl
from jax.experimental.pallas import tpu as pltpu
from jax.experimental.pallas import tpu_sc as plsc

EMB = 32
BATCH = 16384
CHUNK = 128
LANES = 16


@functools.cache
def _build():
    info = plsc.get_sparse_core_info()
    nw = info.num_cores * info.num_subcores
    n = BATCH // nw
    nch = n // CHUNK

    mesh = plsc.VectorSubcoreMesh(core_axis_name="c", subcore_axis_name="s")

    @functools.partial(
        pl.kernel,
        mesh=mesh,
        out_type=jax.ShapeDtypeStruct((2 * BATCH, EMB), jnp.float32),
        compiler_params=pltpu.CompilerParams(use_tc_tiling_on_sc=False),
        scratch_types=[
            pltpu.VMEM((nch, CHUNK), jnp.int32),
            pltpu.VMEM((nch, CHUNK), jnp.int32),
            pltpu.VMEM((nch, CHUNK), jnp.int32),
            pltpu.VMEM((nch, CHUNK), jnp.int32),
            pltpu.VMEM((n, EMB), jnp.float32),
            pltpu.VMEM((n, EMB), jnp.float32),
            pltpu.SemaphoreType.DMA,
        ],
    )
    def k(idx_a_hbm, idx_b_hbm, wa_hbm, wb_hbm, out_hbm,
          ia_v, ib_v, oa_v, ob_v, ra_v, rb_v, sem):
        wid = lax.axis_index("s") * info.num_cores + lax.axis_index("c")
        base = wid * n
        cp_a = pltpu.async_copy(idx_a_hbm.at[pl.ds(wid * nch, nch)], ia_v, sem)
        cp_b = pltpu.async_copy(idx_b_hbm.at[pl.ds(wid * nch, nch)], ib_v, sem)
        iot2 = lax.iota(jnp.int32, LANES) * 2
        for j in range(nch):
            for v in range(CHUNK // LANES):
                s = 2 * (base + j * CHUNK + v * LANES)
                oa_v[j, pl.ds(v * LANES, LANES)] = iot2 + s
                ob_v[j, pl.ds(v * LANES, LANES)] = iot2 + (s + 1)
        cp_a.wait()
        cp_b.wait()
        gathers = []
        for j in range(nch):
            gathers.append(
                pltpu.async_copy(
                    wa_hbm.at[ia_v.at[j]],
                    ra_v.at[pl.ds(j * CHUNK, CHUNK)], sem)
            )
            gathers.append(
                pltpu.async_copy(
                    wb_hbm.at[ib_v.at[j]],
                    rb_v.at[pl.ds(j * CHUNK, CHUNK)], sem)
            )
        for c in gathers:
            c.wait()
        scatters = []
        for j in range(nch):
            scatters.append(
                pltpu.async_copy(ra_v.at[pl.ds(j * CHUNK, CHUNK)], out_hbm.at[oa_v.at[j]], sem)
            )
            scatters.append(
                pltpu.async_copy(rb_v.at[pl.ds(j * CHUNK, CHUNK)], out_hbm.at[ob_v.at[j]], sem)
            )
        for c in scatters:
            c.wait()

    return k


def kernel(field_a, field_b, W_a, W_b):
    k = _build()
    ia = field_a.reshape(BATCH // CHUNK, CHUNK)
    ib = field_b.reshape(BATCH // CHUNK, CHUNK)
    out2 = k(ia, ib, W_a, W_b)
    return out2.reshape(BATCH, 2 * EMB)
